# baseline (speedup 1.0000x reference)
import functools

import jax
import jax.numpy as jnp
from jax import lax
from jax.experimental import pallas as pl
from jax.experimental.pallas import tpu as pltpu
from jax.experimental.pallas import tpu_sc as plsc

_CHUNK = 64
_NBUF = 2


def kernel(x, W):
    B = x.shape[0]
    N, D = W.shape
    info = plsc.get_sparse_core_info()
    NC, NS = info.num_cores, info.num_subcores
    NW = NC * NS
    rows_per_w = N // NW
    n_chunks = rows_per_w // _CHUNK
    mesh = plsc.VectorSubcoreMesh(core_axis_name="c", subcore_axis_name="s")

    @functools.partial(
        pl.kernel,
        mesh=mesh,
        out_type=jax.ShapeDtypeStruct((B, N, D), W.dtype),
        scratch_types=[
            pltpu.VMEM((_NBUF, _CHUNK, D), W.dtype),
            pltpu.SemaphoreType.DMA((_NBUF,)),
            pltpu.SemaphoreType.DMA((_NBUF,)),
        ],
    )
    def run(w_hbm, o_hbm, buf, in_sem, out_sem):
        wid = lax.axis_index("s") * NC + lax.axis_index("c")
        base = wid * rows_per_w

        def in_copy(c, s):
            return pltpu.make_async_copy(
                w_hbm.at[pl.ds(base + c * _CHUNK, _CHUNK), :],
                buf.at[s],
                in_sem.at[s],
            )

        def out_copy(c, s, b):
            return pltpu.make_async_copy(
                buf.at[s],
                o_hbm.at[b, pl.ds(base + c * _CHUNK, _CHUNK), :],
                out_sem.at[s],
            )

        in_copy(0, 0).start()
        for c in range(n_chunks):
            s = c % _NBUF
            if c + 1 < n_chunks:
                nxt = (c + 1) % _NBUF
                if c + 1 >= _NBUF:
                    for b in range(B):
                        out_copy(c + 1 - _NBUF, nxt, b).wait()
                in_copy(c + 1, nxt).start()
            in_copy(c, s).wait()
            for b in range(B):
                out_copy(c, s, b).start()
        for c in range(max(n_chunks - _NBUF, 0), n_chunks):
            for b in range(B):
                out_copy(c, c % _NBUF, b).wait()

    return run(W)
